# baseline (speedup 1.0000x reference)
import jax
import jax.numpy as jnp
from jax import lax
from jax.experimental import pallas as pl
from jax.experimental.pallas import tpu as pltpu
from jax.experimental.pallas import tpu_sc as plsc

NC = 2
NS = 16
L = 16
NW = NC * NS
B = 16384
K = 32
BPW = B // NW
CH = 128
NCH = BPW // CH


def _body(u_hbm, i_hbm, ut_hbm, it_hbm, out_hbm,
          u_idx, i_idx, u_rows, i_rows, out_v, gsem):
    wid = lax.axis_index("s") * NC + lax.axis_index("c")
    base = wid * BPW

    pltpu.sync_copy(u_hbm.at[pl.ds(base, BPW)], u_idx)
    pltpu.sync_copy(i_hbm.at[pl.ds(base, BPW)], i_idx)

    copies = []
    for j in range(NCH):
        sl = pl.ds(j * CH, CH)
        copies.append(pltpu.make_async_copy(
            ut_hbm.at[u_idx.at[sl]], u_rows.at[sl], gsem))
        copies.append(pltpu.make_async_copy(
            it_hbm.at[i_idx.at[sl]], i_rows.at[sl], gsem))
    for c in copies:
        c.start()
    for c in copies:
        c.wait()

    iota = lax.iota(jnp.int32, L)

    def group(g, carry):
        rows = g * L + iota
        acc = jnp.zeros((L,), jnp.float32)
        for k in range(K):
            kk = jnp.full((L,), k, jnp.int32)
            uv = plsc.load_gather(u_rows, [rows, kk])
            iv = plsc.load_gather(i_rows, [rows, kk])
            acc = acc + uv * iv
        out_v[pl.ds(g * L, L)] = acc
        return carry

    lax.fori_loop(0, BPW // L, group, 0)

    pltpu.sync_copy(out_v, out_hbm.at[pl.ds(base, BPW)])


_sc_call = pl.kernel(
    _body,
    out_type=jax.ShapeDtypeStruct((B,), jnp.float32),
    mesh=plsc.VectorSubcoreMesh(
        core_axis_name="c", subcore_axis_name="s",
        num_cores=NC, num_subcores=NS),
    scratch_types=[
        pltpu.VMEM((BPW,), jnp.int32),
        pltpu.VMEM((BPW,), jnp.int32),
        pltpu.VMEM((BPW, K), jnp.float32),
        pltpu.VMEM((BPW, K), jnp.float32),
        pltpu.VMEM((BPW,), jnp.float32),
        pltpu.SemaphoreType.DMA,
    ],
    compiler_params=pltpu.CompilerParams(use_tc_tiling_on_sc=False, needs_layout_passes=False),
)


def kernel(u, i, user_table, item_table):
    return _sc_call(u, i, user_table, item_table)


# speedup vs baseline: 4.6362x; 4.6362x over previous
import jax
import jax.numpy as jnp
from jax import lax
from jax.experimental import pallas as pl
from jax.experimental.pallas import tpu as pltpu
from jax.experimental.pallas import tpu_sc as plsc

NC = 2
NS = 16
L = 16
NW = NC * NS
B = 16384
K = 32
BPW = B // NW
WAVE = 4
NWAVES = BPW // WAVE


def _body(u_hbm, i_hbm, ut_hbm, it_hbm, out_hbm,
          u_vm, i_vm, u_tiles, i_tiles, out_v, gsem):
    wid = lax.axis_index("s") * NC + lax.axis_index("c")
    base = wid * BPW

    pltpu.sync_copy(u_hbm.at[pl.ds(base, BPW)], u_vm.at[pl.ds(0, BPW)])
    pltpu.sync_copy(i_hbm.at[pl.ds(base, BPW)], i_vm.at[pl.ds(0, BPW)])

    def fire(g, slot):
        uvec = u_vm[pl.ds(g * WAVE, L)]
        ivec = i_vm[pl.ds(g * WAVE, L)]
        for j in range(WAVE):
            cu = uvec[j]
            ci = ivec[j]
            cu0 = pl.multiple_of((cu >> 7) << 7, 128)
            ci0 = pl.multiple_of((ci >> 7) << 7, 128)
            pltpu.make_async_copy(
                ut_hbm.at[:, pl.ds(cu0, 128)],
                u_tiles.at[slot, j], gsem).start()
            pltpu.make_async_copy(
                it_hbm.at[:, pl.ds(ci0, 128)],
                i_tiles.at[slot, j], gsem).start()

    def drain(slot):
        for j in range(WAVE):
            pltpu.make_async_copy(
                ut_hbm.at[:, pl.ds(0, 128)],
                u_tiles.at[slot, j], gsem).wait()
            pltpu.make_async_copy(
                it_hbm.at[:, pl.ds(0, 128)],
                i_tiles.at[slot, j], gsem).wait()

    iota = lax.iota(jnp.int32, L)
    lane = iota

    fire(0, 0)

    def step(g, dots):
        slot = lax.rem(g, 2)

        @pl.when(g < NWAVES - 1)
        def _():
            fire(g + 1, 1 - slot)

        drain(slot)

        sv = jnp.full((L,), slot, jnp.int32)
        uvec = u_vm[pl.ds(g * WAVE, L)] & 127
        ivec = i_vm[pl.ds(g * WAVE, L)] & 127
        for j in range(WAVE):
            r = g * WAVE + j
            rr_u = jnp.full((L,), uvec[j], jnp.int32)
            rr_i = jnp.full((L,), ivec[j], jnp.int32)
            jv = jnp.full((L,), j, jnp.int32)
            uv0 = plsc.load_gather(u_tiles, [sv, jv, iota, rr_u])
            uv1 = plsc.load_gather(u_tiles, [sv, jv, iota + L, rr_u])
            iv0 = plsc.load_gather(i_tiles, [sv, jv, iota, rr_i])
            iv1 = plsc.load_gather(i_tiles, [sv, jv, iota + L, rr_i])
            s = uv0 * iv0 + uv1 * iv1
            dot = jnp.sum(s)
            dots = jnp.where(lane == lax.rem(r, L), dot, dots)

        @pl.when(lax.rem(g, 4) == 3)
        def _():
            out_v[pl.ds((g - 3) * WAVE, L)] = dots

        return jnp.where(lax.rem(g, 4) == 3, jnp.zeros((L,), jnp.float32),
                         dots)

    lax.fori_loop(0, NWAVES, step, jnp.zeros((L,), jnp.float32))

    pltpu.sync_copy(out_v, out_hbm.at[pl.ds(base, BPW)])


_sc_call = pl.kernel(
    _body,
    out_type=jax.ShapeDtypeStruct((B,), jnp.float32),
    mesh=plsc.VectorSubcoreMesh(
        core_axis_name="c", subcore_axis_name="s",
        num_cores=NC, num_subcores=NS),
    scratch_types=[
        pltpu.VMEM((BPW + L,), jnp.int32),
        pltpu.VMEM((BPW + L,), jnp.int32),
        pltpu.VMEM((2, WAVE, K, 128), jnp.float32),
        pltpu.VMEM((2, WAVE, K, 128), jnp.float32),
        pltpu.VMEM((BPW,), jnp.float32),
        pltpu.SemaphoreType.DMA,
    ],
    compiler_params=pltpu.CompilerParams(needs_layout_passes=False),
)


def kernel(u, i, user_table, item_table):
    return _sc_call(u, i, user_table.T, item_table.T)
